# baseline (speedup 1.0000x reference)
import functools

import jax
import jax.numpy as jnp
from jax import lax
from jax.experimental import pallas as pl
from jax.experimental.pallas import tpu as pltpu
from jax.experimental.pallas import tpu_sc as plsc

_LANES = 16

_mesh = plsc.VectorSubcoreMesh(core_axis_name="c", subcore_axis_name="s")


@functools.partial(
    pl.kernel,
    mesh=_mesh,
    compiler_params=pltpu.CompilerParams(use_tc_tiling_on_sc=False),
    out_type=jax.ShapeDtypeStruct((_LANES,), jnp.float32),
    scratch_types=[
        pltpu.VMEM((2, 8), jnp.int32),
        pltpu.VMEM((8, 32), jnp.float32),
        pltpu.VMEM((8, 32), jnp.float32),
        pltpu.VMEM((_LANES,), jnp.float32),
        pltpu.SemaphoreType.DMA,
    ],
)
def _sc_dot(idx_hbm, ptab_hbm, utab_hbm, out_hbm, idx_v, prow_v, urow_v, out_v, sem):
    wid = lax.axis_index("s") * 2 + lax.axis_index("c")

    @pl.when(wid == 0)
    def _():
        pltpu.sync_copy(idx_hbm, idx_v)
        cp_p = pltpu.async_copy(ptab_hbm.at[idx_v.at[0]], prow_v, sem)
        cp_u = pltpu.async_copy(utab_hbm.at[idx_v.at[1]], urow_v, sem)
        cp_p.wait()
        cp_u.wait()
        a0 = prow_v[0, pl.ds(0, _LANES)]
        a1 = prow_v[0, pl.ds(_LANES, _LANES)]
        b0 = urow_v[0, pl.ds(0, _LANES)]
        b1 = urow_v[0, pl.ds(_LANES, _LANES)]
        acc = a0 * b0 + a1 * b1
        s = acc[0]
        for i in range(1, _LANES):
            s = s + acc[i]
        out_v[...] = jnp.full((_LANES,), s, jnp.float32)
        pltpu.sync_copy(out_v, out_hbm)


def kernel(product, user, product_embedding, user_embedding):
    p = jnp.asarray(product, jnp.int32)
    u = jnp.asarray(user, jnp.int32)
    idx = jnp.stack([jnp.broadcast_to(p, (8,)), jnp.broadcast_to(u, (8,))])
    out = _sc_dot(idx, product_embedding, user_embedding)
    return out[0]


# speedup vs baseline: 1.4904x; 1.4904x over previous
import functools

import jax
import jax.numpy as jnp
from jax import lax
from jax.experimental import pallas as pl
from jax.experimental.pallas import tpu as pltpu
from jax.experimental.pallas import tpu_sc as plsc

_LANES = 16

_mesh = plsc.VectorSubcoreMesh(core_axis_name="c", subcore_axis_name="s")


@functools.partial(
    pl.kernel,
    mesh=_mesh,
    out_type=jax.ShapeDtypeStruct((_LANES,), jnp.float32),
    scratch_types=[
        pltpu.VMEM((_LANES,), jnp.int32),
        pltpu.VMEM((1, 32), jnp.float32),
        pltpu.VMEM((1, 32), jnp.float32),
        pltpu.VMEM((_LANES,), jnp.float32),
        pltpu.SemaphoreType.DMA,
    ],
)
def _sc_dot(idx_hbm, ptab_hbm, utab_hbm, out_hbm, idx_v, prow_v, urow_v, out_v, sem):
    wid = lax.axis_index("s") * 2 + lax.axis_index("c")

    @pl.when(wid == 0)
    def _():
        pltpu.sync_copy(idx_hbm, idx_v)
        iv = idx_v[...]
        pidx = iv[0]
        uidx = iv[1]
        cp_p = pltpu.async_copy(ptab_hbm.at[pl.ds(pidx, 1)], prow_v, sem)
        cp_u = pltpu.async_copy(utab_hbm.at[pl.ds(uidx, 1)], urow_v, sem)
        cp_p.wait()
        cp_u.wait()
        a0 = prow_v[0, pl.ds(0, _LANES)]
        a1 = prow_v[0, pl.ds(_LANES, _LANES)]
        b0 = urow_v[0, pl.ds(0, _LANES)]
        b1 = urow_v[0, pl.ds(_LANES, _LANES)]
        acc = a0 * b0 + a1 * b1
        s = acc[0]
        for i in range(1, _LANES):
            s = s + acc[i]
        out_v[...] = jnp.full((_LANES,), s, jnp.float32)
        pltpu.sync_copy(out_v, out_hbm)


def kernel(product, user, product_embedding, user_embedding):
    p = jnp.asarray(product, jnp.int32)
    u = jnp.asarray(user, jnp.int32)
    idx = jnp.zeros((_LANES,), jnp.int32).at[0].set(p).at[1].set(u)
    out = _sc_dot(idx, product_embedding, user_embedding)
    return out[0]


# speedup vs baseline: 1.5149x; 1.0165x over previous
import functools

import jax
import jax.numpy as jnp
from jax import lax
from jax.experimental import pallas as pl
from jax.experimental.pallas import tpu as pltpu
from jax.experimental.pallas import tpu_sc as plsc

_LANES = 16

_mesh = plsc.VectorSubcoreMesh(core_axis_name="c", subcore_axis_name="s")


@functools.partial(
    pl.kernel,
    mesh=_mesh,
    compiler_params=pltpu.CompilerParams(skip_device_barrier=True),
    out_type=jax.ShapeDtypeStruct((_LANES,), jnp.float32),
    scratch_types=[
        pltpu.VMEM((_LANES,), jnp.int32),
        pltpu.VMEM((1, 32), jnp.float32),
        pltpu.VMEM((1, 32), jnp.float32),
        pltpu.VMEM((_LANES,), jnp.float32),
        pltpu.SemaphoreType.DMA,
    ],
)
def _sc_dot(idx_hbm, ptab_hbm, utab_hbm, out_hbm, idx_v, prow_v, urow_v, out_v, sem):
    wid = lax.axis_index("s") * 2 + lax.axis_index("c")

    @pl.when(wid == 0)
    def _():
        pltpu.sync_copy(idx_hbm, idx_v)
        iv = idx_v[...]
        pidx = iv[0]
        uidx = iv[1]
        cp_p = pltpu.async_copy(ptab_hbm.at[pl.ds(pidx, 1)], prow_v, sem)
        cp_u = pltpu.async_copy(utab_hbm.at[pl.ds(uidx, 1)], urow_v, sem)
        cp_p.wait()
        cp_u.wait()
        a0 = prow_v[0, pl.ds(0, _LANES)]
        a1 = prow_v[0, pl.ds(_LANES, _LANES)]
        b0 = urow_v[0, pl.ds(0, _LANES)]
        b1 = urow_v[0, pl.ds(_LANES, _LANES)]
        acc = a0 * b0 + a1 * b1
        s = acc[0]
        for i in range(1, _LANES):
            s = s + acc[i]
        out_v[...] = jnp.full((_LANES,), s, jnp.float32)
        pltpu.sync_copy(out_v, out_hbm)


def kernel(product, user, product_embedding, user_embedding):
    p = jnp.asarray(product, jnp.int32)
    u = jnp.asarray(user, jnp.int32)
    idx = jnp.zeros((_LANES,), jnp.int32).at[0].set(p).at[1].set(u)
    out = _sc_dot(idx, product_embedding, user_embedding)
    return out[0]
